# baseline (speedup 1.0000x reference)
import jax
import jax.numpy as jnp
from jax import lax
from jax.experimental import pallas as pl
from jax.experimental.pallas import tpu as pltpu
from jax.experimental.pallas import tpu_sc as plsc

N_NODES = 10000
N_EDGES = 160000
D_FEAT = 256
DQ = 64
DW = DQ // 2
NQ = D_FEAT // DQ
NPASS = NQ // 2

NC = 2
NS = 16
CHUNK = 128
NCHUNKS = 80
EPT = NCHUNKS * CHUNK
E_PAD = NS * EPT
NBUF = 4
NF = 2
NIDX = 8
E_EXTRA = NIDX * CHUNK

ACC_ROWS = 10112
ZROWS = ACC_ROWS // NS
WROWS = 624
WROWS_LAST = N_NODES - 15 * WROWS


def _sc_body(xq_hbm, src_hbm, dst_hbm, zeros_hbm, out_hbm,
             src_vs, dst_vs, rowsp_vs, rowsf_vs, xtab, acc, *sems):
    c = lax.axis_index("c")
    s = lax.axis_index("s")
    ebase = s * EPT

    src_v = [src_vs.at[j] for j in range(NIDX)]
    dst_v = [dst_vs.at[j] for j in range(NIDX)]
    rowsp = [rowsp_vs.at[b] for b in range(NBUF)]
    rowsf = [rowsf_vs.at[b] for b in range(NF)]
    semg = list(sems[0:NBUF])
    semsc = list(sems[NBUF:NBUF + NF])
    semi = list(sems[NBUF + NF:NBUF + NF + NIDX])

    def idx_start(k, j):
        base = ebase + k * CHUNK
        pltpu.async_copy(src_hbm.at[pl.ds(base, CHUNK)], src_v[j], semi[j])
        pltpu.async_copy(dst_hbm.at[pl.ds(base, CHUNK)], dst_v[j], semi[j])

    def idx_wait(k, j):
        base = ebase + k * CHUNK
        pltpu.make_async_copy(src_hbm.at[pl.ds(base, CHUNK)], src_v[j],
                              semi[j]).wait()
        pltpu.make_async_copy(dst_hbm.at[pl.ds(base, CHUNK)], dst_v[j],
                              semi[j]).wait()

    def startg(j, b):
        pltpu.async_copy(xtab.at[src_v[j]], rowsp[b], semg[b])

    def waitg(j, b):
        pltpu.make_async_copy(xtab.at[src_v[j]], rowsp[b], semg[b]).wait()

    def scat_start(j, b):
        pltpu.async_copy(rowsf[b], acc.at[dst_v[j]], semsc[b], add=True)

    def scat_wait(j, b):
        pltpu.make_async_copy(rowsf[b], acc.at[dst_v[j]], semsc[b]).wait()

    hi_mask = jnp.int32(-65536)

    def unpack(bg, bf):
        def row_body(r, carry):
            for u in range(DW // 16):
                w = rowsp[bg][r, pl.ds(u * 16, 16)]
                rowsf[bf][r, pl.ds(u * 16, 16)] = lax.bitcast_convert_type(
                    lax.shift_left(w, 16), jnp.float32)
                rowsf[bf][r, pl.ds(DW + u * 16, 16)] = lax.bitcast_convert_type(
                    lax.bitwise_and(w, hi_mask), jnp.float32)
            return carry

        lax.fori_loop(0, CHUNK, row_body, 0)

    for p in range(NPASS):
        q = c * NPASS + p

        @pl.when(s < NS - 1)
        def _():
            pltpu.sync_copy(xq_hbm.at[pl.ds(q * N_NODES + s * WROWS, WROWS)],
                            xtab.at[pl.ds(s * WROWS, WROWS)])

        @pl.when(s == NS - 1)
        def _():
            pltpu.sync_copy(
                xq_hbm.at[pl.ds(q * N_NODES + 15 * WROWS, WROWS_LAST)],
                xtab.at[pl.ds(15 * WROWS, WROWS_LAST)])

        pltpu.sync_copy(zeros_hbm, acc.at[pl.ds(s * ZROWS, ZROWS)])
        plsc.subcore_barrier()

        for j in range(NIDX):
            idx_start(j, j)
        for b in range(NBUF):
            idx_wait(b, b)
            startg(b, b)

        def group(k, first):
            for d in range(NIDX):
                bg = d % NBUF
                bf = d % NF
                waitg(d, bg)
                if not (first and d < NF):
                    scat_wait((d - NF) % NIDX, bf)
                    idx_start(k + d + 6, (d - NF) % NIDX)
                unpack(bg, bf)
                scat_start(d, bf)
                idx_wait(k + d + NBUF, (d + NBUF) % NIDX)
                startg((d + NBUF) % NIDX, bg)

        group(0, True)

        def pipe(i, carry):
            group(NIDX * i, False)
            return carry

        lax.fori_loop(1, NCHUNKS // NIDX, pipe, 0)

        scat_wait((NIDX - NF) % NIDX, 0)
        scat_wait((NIDX - 1) % NIDX, 1)
        for b in range(NBUF):
            waitg(b, b)
        idx_wait(NCHUNKS + NBUF, NBUF)
        idx_wait(NCHUNKS + NBUF + 1, NBUF + 1)

        plsc.subcore_barrier()

        @pl.when(s < NS - 1)
        def _():
            pltpu.sync_copy(acc.at[pl.ds(s * WROWS, WROWS)],
                            out_hbm.at[pl.ds(q * N_NODES + s * WROWS, WROWS)])

        @pl.when(s == NS - 1)
        def _():
            pltpu.sync_copy(
                acc.at[pl.ds(15 * WROWS, WROWS_LAST)],
                out_hbm.at[pl.ds(q * N_NODES + 15 * WROWS, WROWS_LAST)])

        if p + 1 < NPASS:
            plsc.subcore_barrier()


def kernel(x, edge_index):
    xb = x.astype(jnp.bfloat16).reshape(N_NODES, NQ, 2, DW)
    xpairs = xb.transpose(1, 0, 3, 2)
    xqp = lax.bitcast_convert_type(xpairs, jnp.int32).reshape(NQ * N_NODES, DW)
    src = edge_index[0].astype(jnp.int32)
    dst = edge_index[1].astype(jnp.int32)
    pad = E_PAD + E_EXTRA - N_EDGES
    src_p = jnp.concatenate([src, jnp.zeros((pad,), jnp.int32)])
    dst_p = jnp.concatenate([dst, jnp.full((pad,), N_NODES, jnp.int32)])
    zeros = jnp.zeros((ZROWS, DQ), jnp.float32)

    mesh = plsc.VectorSubcoreMesh(core_axis_name="c", subcore_axis_name="s",
                                  num_cores=NC, num_subcores=NS)
    out = pl.kernel(
        _sc_body,
        out_type=jax.ShapeDtypeStruct((NQ * N_NODES, DQ), jnp.float32),
        mesh=mesh,
        compiler_params=pltpu.CompilerParams(use_tc_tiling_on_sc=False),
        scratch_types=[
            pltpu.VMEM((NIDX, CHUNK), jnp.int32),
            pltpu.VMEM((NIDX, CHUNK), jnp.int32),
            pltpu.VMEM((NBUF, CHUNK, DW), jnp.int32),
            pltpu.VMEM((NF, CHUNK, DQ), jnp.float32),
            pltpu.VMEM_SHARED((N_NODES, DW), jnp.int32),
            pltpu.VMEM_SHARED((ACC_ROWS, DQ), jnp.float32),
        ] + [pltpu.SemaphoreType.DMA] * (NBUF + NF + NIDX),
    )(xqp, src_p, dst_p, zeros)

    return out.reshape(NQ, N_NODES, DQ).transpose(1, 0, 2).reshape(N_NODES, D_FEAT)


# speedup vs baseline: 1.4150x; 1.4150x over previous
import jax
import jax.numpy as jnp
from jax import lax
from jax.experimental import pallas as pl
from jax.experimental.pallas import tpu as pltpu
from jax.experimental.pallas import tpu_sc as plsc

N_NODES = 10000
N_EDGES = 160000
D_FEAT = 256
DQ = 64
DW = 64
NQ = D_FEAT // DQ

NC = 2
NS = 16
CHUNK = 24
NCHUNKS = 420
EPT = NCHUNKS * CHUNK
E_PAD = NS * EPT
NBUF = 2
NIDX = 6
E_EXTRA = NIDX * CHUNK

ACC_ROWS = 10112
ZROWS = ACC_ROWS // NS
WROWS = 624
WROWS_LAST = N_NODES - 15 * WROWS


def _sc_body(xp_hbm, src_hbm, dst_hbm, zeros_hbm, out_hbm,
             src_vs, dst_vs, rowsp_vs, lof_vs, hif_vs, xtab, acca, accb,
             *sems):
    c = lax.axis_index("c")
    s = lax.axis_index("s")
    ebase = s * EPT

    src_v = [src_vs.at[j] for j in range(NIDX)]
    dst_v = [dst_vs.at[j] for j in range(NIDX)]
    rowsp = [rowsp_vs.at[b] for b in range(NBUF)]
    lof = [lof_vs.at[b] for b in range(NBUF)]
    hif = [hif_vs.at[b] for b in range(NBUF)]
    semg = list(sems[0:NBUF])
    semsa = list(sems[NBUF:2 * NBUF])
    semsb = list(sems[2 * NBUF:3 * NBUF])
    semi = list(sems[3 * NBUF:3 * NBUF + NIDX])

    def idx_start(k, j):
        base = ebase + k * CHUNK
        pltpu.async_copy(src_hbm.at[pl.ds(base, CHUNK)], src_v[j], semi[j])
        pltpu.async_copy(dst_hbm.at[pl.ds(base, CHUNK)], dst_v[j], semi[j])

    def idx_wait(k, j):
        base = ebase + k * CHUNK
        pltpu.make_async_copy(src_hbm.at[pl.ds(base, CHUNK)], src_v[j],
                              semi[j]).wait()
        pltpu.make_async_copy(dst_hbm.at[pl.ds(base, CHUNK)], dst_v[j],
                              semi[j]).wait()

    def startg(j, b):
        pltpu.async_copy(xtab.at[src_v[j]], rowsp[b], semg[b])

    def waitg(j, b):
        pltpu.make_async_copy(xtab.at[src_v[j]], rowsp[b], semg[b]).wait()

    def scat_start(j, b):
        pltpu.async_copy(lof[b], acca.at[dst_v[j]], semsa[b], add=True)
        pltpu.async_copy(hif[b], accb.at[dst_v[j]], semsb[b], add=True)

    def scat_wait(j, b):
        pltpu.make_async_copy(lof[b], acca.at[dst_v[j]], semsa[b]).wait()
        pltpu.make_async_copy(hif[b], accb.at[dst_v[j]], semsb[b]).wait()

    hi_mask = jnp.int32(-65536)

    def unpack(b):
        def row_body(r, carry):
            for u in range(DW // 16):
                w = rowsp[b][r, pl.ds(u * 16, 16)]
                lof[b][r, pl.ds(u * 16, 16)] = lax.bitcast_convert_type(
                    lax.shift_left(w, 16), jnp.float32)
                hif[b][r, pl.ds(u * 16, 16)] = lax.bitcast_convert_type(
                    lax.bitwise_and(w, hi_mask), jnp.float32)
            return carry

        lax.fori_loop(0, CHUNK, row_body, 0)

    @pl.when(s < NS - 1)
    def _():
        pltpu.sync_copy(xp_hbm.at[pl.ds(c * N_NODES + s * WROWS, WROWS)],
                        xtab.at[pl.ds(s * WROWS, WROWS)])

    @pl.when(s == NS - 1)
    def _():
        pltpu.sync_copy(
            xp_hbm.at[pl.ds(c * N_NODES + 15 * WROWS, WROWS_LAST)],
            xtab.at[pl.ds(15 * WROWS, WROWS_LAST)])

    pltpu.sync_copy(zeros_hbm, acca.at[pl.ds(s * ZROWS, ZROWS)])
    pltpu.sync_copy(zeros_hbm, accb.at[pl.ds(s * ZROWS, ZROWS)])
    plsc.subcore_barrier()

    for j in range(NIDX - 2):
        idx_start(j, j)
    for b in range(NBUF):
        idx_wait(b, b)
        startg(b, b)

    def group(k, first):
        for d in range(NIDX):
            b = d % NBUF
            waitg(d, b)
            if not (first and d < NBUF):
                scat_wait((d - NBUF) % NIDX, b)
            idx_start(k + d + NIDX - 2, (d - 2) % NIDX)
            unpack(b)
            scat_start(d, b)
            idx_wait(k + d + 2, (d + 2) % NIDX)
            startg((d + 2) % NIDX, b)

    group(0, True)

    def pipe(i, carry):
        group(NIDX * i, False)
        return carry

    lax.fori_loop(1, NCHUNKS // NIDX, pipe, 0)

    scat_wait(NIDX - 2, 0)
    scat_wait(NIDX - 1, 1)
    waitg(0, 0)
    waitg(1, 1)
    idx_wait(NCHUNKS + 2, 2)
    idx_wait(NCHUNKS + 3, 3)

    plsc.subcore_barrier()

    for acc, qoff in ((acca, 0), (accb, 1)):
        q = NC * c + qoff

        @pl.when(s < NS - 1)
        def _(acc=acc, q=q):
            pltpu.sync_copy(acc.at[pl.ds(s * WROWS, WROWS)],
                            out_hbm.at[pl.ds(q * N_NODES + s * WROWS, WROWS)])

        @pl.when(s == NS - 1)
        def _(acc=acc, q=q):
            pltpu.sync_copy(
                acc.at[pl.ds(15 * WROWS, WROWS_LAST)],
                out_hbm.at[pl.ds(q * N_NODES + 15 * WROWS, WROWS_LAST)])


def kernel(x, edge_index):
    xb = x.astype(jnp.bfloat16).reshape(N_NODES, NC, 2, DW)
    xpairs = xb.transpose(1, 0, 3, 2)
    xp = lax.bitcast_convert_type(xpairs, jnp.int32).reshape(NC * N_NODES, DW)
    src = edge_index[0].astype(jnp.int32)
    dst = edge_index[1].astype(jnp.int32)
    pad = E_PAD + E_EXTRA - N_EDGES
    src_p = jnp.concatenate([src, jnp.zeros((pad,), jnp.int32)])
    dst_p = jnp.concatenate([dst, jnp.full((pad,), N_NODES, jnp.int32)])
    zeros = jnp.zeros((ZROWS, DQ), jnp.float32)

    mesh = plsc.VectorSubcoreMesh(core_axis_name="c", subcore_axis_name="s",
                                  num_cores=NC, num_subcores=NS)
    out = pl.kernel(
        _sc_body,
        out_type=jax.ShapeDtypeStruct((NQ * N_NODES, DQ), jnp.float32),
        mesh=mesh,
        compiler_params=pltpu.CompilerParams(use_tc_tiling_on_sc=False),
        scratch_types=[
            pltpu.VMEM((NIDX, CHUNK), jnp.int32),
            pltpu.VMEM((NIDX, CHUNK), jnp.int32),
            pltpu.VMEM((NBUF, CHUNK, DW), jnp.int32),
            pltpu.VMEM((NBUF, CHUNK, DQ), jnp.float32),
            pltpu.VMEM((NBUF, CHUNK, DQ), jnp.float32),
            pltpu.VMEM_SHARED((N_NODES, DW), jnp.int32),
            pltpu.VMEM_SHARED((ACC_ROWS, DQ), jnp.float32),
            pltpu.VMEM_SHARED((ACC_ROWS, DQ), jnp.float32),
        ] + [pltpu.SemaphoreType.DMA] * (3 * NBUF + NIDX),
    )(xp, src_p, dst_p, zeros)

    return out.reshape(NQ, N_NODES, DQ).transpose(1, 0, 2).reshape(N_NODES, D_FEAT)
